# Initial kernel scaffold; baseline (speedup 1.0000x reference)
import jax
import jax.numpy as jnp
from jax.experimental import pallas as pl


def kernel(x, edge_index, W_proj, b_proj, g1, be1, W1, as1, ad1, bc1, g2, be2, W2, as2, ad2, bc2, g3, be3, W_cls, b_cls):
    raise NotImplementedError("write your pallas kernel here")


# speedup vs baseline: 52.8430x; 52.8430x over previous
import functools

import jax
import jax.numpy as jnp
from jax import lax
from jax.experimental import pallas as pl
from jax.experimental.pallas import tpu as pltpu
from jax.experimental.pallas import tpu_sc as plsc

N = 10000
D = 128
NHEADS1 = 8
E = 320000
ETOT = E + N

NC = 2
NS = 16
NW = NC * NS

NPAD = 10112
ROWS_PER_TILE = NPAD // NS

TCOLS = 144
ACOLS = 16

C = 192
PER_W = 10368
NCHUNK = PER_W // C
EPAD = PER_W * NW

_SENT = -1e30


def _head_expand_mask(heads, oc):
    r = lax.broadcasted_iota(jnp.int32, (heads, 128), 0)
    c = lax.broadcasted_iota(jnp.int32, (heads, 128), 1) // oc
    return (r == c).astype(jnp.float32)


@functools.lru_cache(maxsize=None)
def _make_edge_kernel(heads):
    oc = 128 // heads
    mesh = plsc.VectorSubcoreMesh(core_axis_name="c", subcore_axis_name="s",
                                  num_cores=NC, num_subcores=NS)

    @functools.partial(
        pl.kernel,
        mesh=mesh,
        compiler_params=pltpu.CompilerParams(use_tc_tiling_on_sc=False,
                                             needs_layout_passes=False),
        out_type=jax.ShapeDtypeStruct((NC, NPAD, TCOLS), jnp.float32),
        scratch_types=[
            pltpu.VMEM((C,), jnp.int32),
            pltpu.VMEM((C,), jnp.int32),
            pltpu.VMEM((C, TCOLS), jnp.float32),
            pltpu.VMEM((C, ACOLS), jnp.float32),
            pltpu.VMEM_SHARED((NPAD, TCOLS), jnp.float32),
            pltpu.SemaphoreType.DMA,
            pltpu.SemaphoreType.DMA,
        ],
    )
    def edge_kernel(tab_hbm, adst_hbm, src_hbm, dst_hbm, out_hbm,
                    src_c, dst_c, rows, adst_c, acc, sem, sem2):
        cid = lax.axis_index("c")
        sid = lax.axis_index("s")
        wid = cid * NS + sid

        def _zrow(i, _):
            for k in range(TCOLS // 16):
                rows[i, pl.ds(k * 16, 16)] = jnp.zeros((16,), jnp.float32)
            return 0
        lax.fori_loop(0, C, _zrow, 0)
        r0 = sid * ROWS_PER_TILE
        for off in range(0, ROWS_PER_TILE, C):
            nrow = min(C, ROWS_PER_TILE - off)
            pltpu.sync_copy(rows.at[pl.ds(0, nrow)],
                            acc.at[pl.ds(r0 + off, nrow)])
        plsc.subcore_barrier()

        iota16 = lax.iota(jnp.int32, 16)

        def chunk_body(ch, _):
            base = wid * PER_W + ch * C
            pltpu.sync_copy(src_hbm.at[pl.ds(base, C)], src_c)
            pltpu.sync_copy(dst_hbm.at[pl.ds(base, C)], dst_c)
            cp1 = pltpu.async_copy(tab_hbm.at[src_c], rows, sem)
            cp2 = pltpu.async_copy(adst_hbm.at[dst_c], adst_c, sem2)
            cp1.wait()
            cp2.wait()

            def jbody(j, _):
                rb = j * 16 + iota16
                for h in range(heads):
                    colv = jnp.full((16,), 128 + h, jnp.int32)
                    av = plsc.load_gather(rows, [rb, colv])
                    dv = plsc.load_gather(adst_c, [rb, jnp.full((16,), h, jnp.int32)])
                    a = av + dv
                    a = jnp.maximum(a, a * jnp.float32(0.2))
                    plsc.store_scatter(rows, [rb, colv], jnp.exp(a))
                return 0
            lax.fori_loop(0, C // 16, jbody, 0)

            def ebody(e, _):
                exv = rows[e, pl.ds(128, 16)]
                for h in range(heads):
                    s = exv[h]
                    for cc in range(oc // 16):
                        col = h * oc + cc * 16
                        rows[e, pl.ds(col, 16)] = rows[e, pl.ds(col, 16)] * s
                return 0
            lax.fori_loop(0, C, ebody, 0)

            pltpu.sync_copy(rows, acc.at[dst_c], add=True)
            return 0
        lax.fori_loop(0, NCHUNK, chunk_body, 0)

        plsc.subcore_barrier()
        pltpu.sync_copy(acc.at[pl.ds(r0, ROWS_PER_TILE)],
                        out_hbm.at[cid, pl.ds(r0, ROWS_PER_TILE)])

    return edge_kernel


def _bn(h, g, b):
    mu = jnp.mean(h, axis=0, keepdims=True)
    var = jnp.mean((h - mu) ** 2, axis=0, keepdims=True)
    return g[None, :] * (h - mu) / jnp.sqrt(var + 1e-5) + b[None, :]


def _elu(h):
    return jnp.where(h > 0, h, jnp.exp(jnp.minimum(h, 0.0)) - 1.0)


def _stage_a_body(x_ref, wp_ref, bp_ref, g1_ref, be1_ref, w1_ref, as1_ref,
                  ad1_ref, hp_ref, t1_ref, adt1_ref):
    x = x_ref[...]
    h0 = jnp.dot(x, wp_ref[...], preferred_element_type=jnp.float32)
    h0 = h0 + bp_ref[...][None, :]
    hp = _elu(_bn(h0, g1_ref[...], be1_ref[...]))
    hp_ref[...] = hp
    h1 = jnp.dot(hp, w1_ref[...], preferred_element_type=jnp.float32)
    m = _head_expand_mask(NHEADS1, 128 // NHEADS1)
    a_s = as1_ref[...]
    a_d = ad1_ref[...]
    asrc = jnp.dot(h1, (m * a_s[None, :]).T, preferred_element_type=jnp.float32)
    adst = jnp.dot(h1, (m * a_d[None, :]).T, preferred_element_type=jnp.float32)
    zpadN = jnp.zeros((N, TCOLS - 136), jnp.float32)
    body = jnp.concatenate([h1, asrc, zpadN], axis=1)
    sent = jnp.concatenate([
        jnp.zeros((NPAD - N, 128), jnp.float32),
        jnp.full((NPAD - N, 8), _SENT, jnp.float32),
        jnp.zeros((NPAD - N, TCOLS - 136), jnp.float32)], axis=1)
    t1_ref[...] = jnp.concatenate([body, sent], axis=0)
    adt = jnp.concatenate([adst, jnp.zeros((N, ACOLS - 8), jnp.float32)], axis=1)
    adt1_ref[...] = jnp.concatenate(
        [adt, jnp.zeros((NPAD - N, ACOLS), jnp.float32)], axis=0)


def _stage_c_body(acc_ref, hp_ref, g2_ref, be2_ref, bc1_ref, w2_ref, as2_ref,
                  ad2_ref, h2_ref, t2_ref, adt2_ref):
    s = acc_ref[0] + acc_ref[1]
    num = s[0:N, 0:128]
    den8 = s[0:N, 128:136]
    m = _head_expand_mask(NHEADS1, 128 // NHEADS1)
    denf = jnp.dot(den8, m, preferred_element_type=jnp.float32)
    o1 = num / (denf + 1e-16) + bc1_ref[...][None, :]
    h2 = _elu(_bn(o1, g2_ref[...], be2_ref[...]) + hp_ref[...])
    h2_ref[...] = h2
    h2w = jnp.dot(h2, w2_ref[...], preferred_element_type=jnp.float32)
    a_s = as2_ref[...]
    a_d = ad2_ref[...]
    asrc = jnp.dot(h2w, a_s[:, None], preferred_element_type=jnp.float32)
    adst = jnp.dot(h2w, a_d[:, None], preferred_element_type=jnp.float32)
    body = jnp.concatenate(
        [h2w, asrc, jnp.zeros((N, TCOLS - 129), jnp.float32)], axis=1)
    sent = jnp.concatenate([
        jnp.zeros((NPAD - N, 128), jnp.float32),
        jnp.full((NPAD - N, 1), _SENT, jnp.float32),
        jnp.zeros((NPAD - N, TCOLS - 129), jnp.float32)], axis=1)
    t2_ref[...] = jnp.concatenate([body, sent], axis=0)
    adt = jnp.concatenate([adst, jnp.zeros((N, ACOLS - 1), jnp.float32)], axis=1)
    adt2_ref[...] = jnp.concatenate(
        [adt, jnp.zeros((NPAD - N, ACOLS), jnp.float32)], axis=0)


def _stage_e_body(acc_ref, h2_ref, g3_ref, be3_ref, bc2_ref, wc_ref, bcls_ref,
                  out_ref):
    s = acc_ref[0] + acc_ref[1]
    num = s[0:N, 0:128]
    den = s[0:N, 128:129]
    o2 = num / (den + 1e-16) + bc2_ref[...][None, :]
    h3 = _elu(_bn(o2, g3_ref[...], be3_ref[...]) + h2_ref[...])
    out_ref[...] = jnp.dot(h3, wc_ref[...], preferred_element_type=jnp.float32) \
        + bcls_ref[...][None, :]


_stage_a = pl.pallas_call(
    _stage_a_body,
    out_shape=[
        jax.ShapeDtypeStruct((N, D), jnp.float32),
        jax.ShapeDtypeStruct((NPAD, TCOLS), jnp.float32),
        jax.ShapeDtypeStruct((NPAD, ACOLS), jnp.float32),
    ],
)

_stage_c = pl.pallas_call(
    _stage_c_body,
    out_shape=[
        jax.ShapeDtypeStruct((N, D), jnp.float32),
        jax.ShapeDtypeStruct((NPAD, TCOLS), jnp.float32),
        jax.ShapeDtypeStruct((NPAD, ACOLS), jnp.float32),
    ],
)

_stage_e = pl.pallas_call(
    _stage_e_body,
    out_shape=jax.ShapeDtypeStruct((N, 40), jnp.float32),
)


def kernel(x, edge_index, W_proj, b_proj, g1, be1, W1, as1, ad1, bc1,
           g2, be2, W2, as2, ad2, bc2, g3, be3, W_cls, b_cls):
    ei = edge_index.astype(jnp.int32)
    loop = jnp.arange(N, dtype=jnp.int32)
    padi = jnp.full((EPAD - ETOT,), N, jnp.int32)
    src = jnp.concatenate([ei[0], loop, padi])
    dst = jnp.concatenate([ei[1], loop, padi])

    hp, t1, adt1 = _stage_a(x, W_proj, b_proj, g1, be1, W1,
                            as1.reshape(-1), ad1.reshape(-1))
    acc1 = _make_edge_kernel(NHEADS1)(t1, adt1, src, dst)
    h2, t2, adt2 = _stage_c(acc1, hp, g2, be2, bc1, W2,
                            as2.reshape(-1), ad2.reshape(-1))
    acc2 = _make_edge_kernel(1)(t2, adt2, src, dst)
    return _stage_e(acc2, h2, g3, be3, bc2, W_cls, b_cls)
